# Initial kernel scaffold; baseline (speedup 1.0000x reference)
import jax
import jax.numpy as jnp
from jax.experimental import pallas as pl


def kernel(x, edge_index, edge_attr, msg_Win, msg_bin, msg_W1, msg_b1, msg_W2, msg_b2, upd_Win, upd_bin, upd_W1, upd_b1, upd_W2, upd_b2):
    raise NotImplementedError("write your pallas kernel here")


# speedup vs baseline: 3.4277x; 3.4277x over previous
import functools

import jax
import jax.numpy as jnp
from jax import lax
from jax.experimental import pallas as pl
from jax.experimental.pallas import tpu as pltpu
from jax.experimental.pallas import tpu_sc as plsc

N_NODES = 10000
N_EDGES = 320000
D = 128
D_EDGE = 16

NC = 2
NS = 16
NW = NC * NS

CH = 128
N_CHUNKS = N_EDGES // CH
PER_W = -(-N_CHUNKS // NW)
N_PAD = 10240
ROWS_PER_TILE = N_PAD // NS

@functools.cache
def _sc_gather():
    mesh = plsc.VectorSubcoreMesh(core_axis_name="c", subcore_axis_name="s",
                                  num_cores=NC, num_subcores=NS)

    @functools.partial(
        pl.kernel,
        out_type=(
            jax.ShapeDtypeStruct((N_EDGES, D), jnp.float32),
            jax.ShapeDtypeStruct((N_EDGES, D), jnp.float32),
        ),
        mesh=mesh,
        scratch_types=[
            pltpu.VMEM((CH,), jnp.int32),
            pltpu.VMEM((CH, D), jnp.float32),
            pltpu.SemaphoreType.DMA,
        ],
    )
    def body(pa_hbm, pb_hbm, src_hbm, dst_hbm, gs_hbm, gd_hbm,
             idx_v, rows_v, sem):
        c = lax.axis_index("c")
        s = lax.axis_index("s")
        wid = s * NC + c

        @pl.loop(0, PER_W)
        def _(j):
            chunk = wid + j * NW

            @pl.when(chunk < N_CHUNKS)
            def _():
                base = chunk * CH
                pltpu.sync_copy(src_hbm.at[pl.ds(base, CH)], idx_v)
                pltpu.async_copy(pa_hbm.at[idx_v], rows_v, sem).wait()
                pltpu.sync_copy(rows_v, gs_hbm.at[pl.ds(base, CH)])
                pltpu.sync_copy(dst_hbm.at[pl.ds(base, CH)], idx_v)
                pltpu.async_copy(pb_hbm.at[idx_v], rows_v, sem).wait()
                pltpu.sync_copy(rows_v, gd_hbm.at[pl.ds(base, CH)])

    return body


@functools.cache
def _sc_scatter():
    mesh = plsc.VectorSubcoreMesh(core_axis_name="c", subcore_axis_name="s",
                                  num_cores=NC, num_subcores=NS)

    @functools.partial(
        pl.kernel,
        out_type=jax.ShapeDtypeStruct((NC, N_PAD, D), jnp.float32),
        mesh=mesh,
        scratch_types=[
            pltpu.VMEM((CH,), jnp.int32),
            pltpu.VMEM((CH, D), jnp.float32),
            pltpu.VMEM_SHARED((N_PAD, D), jnp.float32),
            pltpu.SemaphoreType.DMA,
        ],
    )
    def body(msg_hbm, dst_hbm, zeros_hbm, part_hbm, idx_v, rows_v, acc, sem):
        c = lax.axis_index("c")
        s = lax.axis_index("s")
        wid = s * NC + c
        rbase = s * ROWS_PER_TILE

        pltpu.sync_copy(zeros_hbm.at[pl.ds(rbase, ROWS_PER_TILE)],
                        acc.at[pl.ds(rbase, ROWS_PER_TILE)])
        plsc.subcore_barrier()

        @pl.loop(0, PER_W)
        def _(j):
            chunk = wid + j * NW

            @pl.when(chunk < N_CHUNKS)
            def _():
                base = chunk * CH
                pltpu.sync_copy(dst_hbm.at[pl.ds(base, CH)], idx_v)
                pltpu.sync_copy(msg_hbm.at[pl.ds(base, CH)], rows_v)
                pltpu.sync_copy(rows_v, acc.at[idx_v], add=True)

        plsc.subcore_barrier()
        pltpu.sync_copy(acc.at[pl.ds(rbase, ROWS_PER_TILE)],
                        part_hbm.at[c, pl.ds(rbase, ROWS_PER_TILE)])

    return body


_NB = 2000
_EB = 2560

_full = lambda shape: pl.BlockSpec(shape, lambda i: (0,) * len(shape))


def _pre_body(x_ref, wa_ref, wb_ref, bin_ref, pa_ref, pb_ref):
    xb = x_ref[...]
    pa_ref[...] = jnp.dot(xb, wa_ref[...], preferred_element_type=jnp.float32)
    pb_ref[...] = (jnp.dot(xb, wb_ref[...], preferred_element_type=jnp.float32)
                   + bin_ref[...])


def _tc_pre(x, wa, wb, b_in):
    return pl.pallas_call(
        _pre_body,
        grid=(N_NODES // _NB,),
        in_specs=[
            pl.BlockSpec((_NB, D), lambda i: (i, 0)),
            _full((D, D)), _full((D, D)), _full((1, D)),
        ],
        out_specs=[
            pl.BlockSpec((_NB, D), lambda i: (i, 0)),
            pl.BlockSpec((_NB, D), lambda i: (i, 0)),
        ],
        out_shape=[
            jax.ShapeDtypeStruct((N_NODES, D), jnp.float32),
            jax.ShapeDtypeStruct((N_NODES, D), jnp.float32),
        ],
    )(x, wa, wb, b_in)


def _msg_body(gs_ref, gd_ref, ea_ref, wc_ref, w1_ref, b1_ref, w2_ref, b2_ref,
              out_ref):
    h = gs_ref[...] + gd_ref[...] + jnp.dot(
        ea_ref[...], wc_ref[...], preferred_element_type=jnp.float32)
    h = jnp.maximum(h, 0.0)
    h = h + jnp.maximum(
        jnp.dot(h, w1_ref[...], preferred_element_type=jnp.float32)
        + b1_ref[...], 0.0)
    out_ref[...] = h + jnp.maximum(
        jnp.dot(h, w2_ref[...], preferred_element_type=jnp.float32)
        + b2_ref[...], 0.0)


def _tc_msg(gs, gd, ea, wc, w1, b1, w2, b2):
    return pl.pallas_call(
        _msg_body,
        grid=(N_EDGES // _EB,),
        in_specs=[
            pl.BlockSpec((_EB, D), lambda i: (i, 0)),
            pl.BlockSpec((_EB, D), lambda i: (i, 0)),
            pl.BlockSpec((_EB, D_EDGE), lambda i: (i, 0)),
            _full((D_EDGE, D)), _full((D, D)), _full((1, D)),
            _full((D, D)), _full((1, D)),
        ],
        out_specs=pl.BlockSpec((_EB, D), lambda i: (i, 0)),
        out_shape=jax.ShapeDtypeStruct((N_EDGES, D), jnp.float32),
    )(gs, gd, ea, wc, w1, b1, w2, b2)


def _upd_body(p0_ref, p1_ref, x_ref, wi_ref, bi_ref, w1_ref, b1_ref,
              w2_ref, b2_ref, out_ref):
    agg = p0_ref[...] + p1_ref[...]
    h = jnp.maximum(
        jnp.dot(agg, wi_ref[...], preferred_element_type=jnp.float32)
        + bi_ref[...], 0.0)
    h = h + jnp.maximum(
        jnp.dot(h, w1_ref[...], preferred_element_type=jnp.float32)
        + b1_ref[...], 0.0)
    h = h + jnp.maximum(
        jnp.dot(h, w2_ref[...], preferred_element_type=jnp.float32)
        + b2_ref[...], 0.0)
    out_ref[...] = x_ref[...] + h


def _tc_upd(p0, p1, x, wi, bi, w1, b1, w2, b2):
    return pl.pallas_call(
        _upd_body,
        grid=(N_NODES // _NB,),
        in_specs=[
            pl.BlockSpec((_NB, D), lambda i: (i, 0)),
            pl.BlockSpec((_NB, D), lambda i: (i, 0)),
            pl.BlockSpec((_NB, D), lambda i: (i, 0)),
            _full((D, D)), _full((1, D)),
            _full((D, D)), _full((1, D)),
            _full((D, D)), _full((1, D)),
        ],
        out_specs=pl.BlockSpec((_NB, D), lambda i: (i, 0)),
        out_shape=jax.ShapeDtypeStruct((N_NODES, D), jnp.float32),
    )(p0, p1, x, wi, bi, w1, b1, w2, b2)


def kernel(x, edge_index, edge_attr,
           msg_Win, msg_bin, msg_W1, msg_b1, msg_W2, msg_b2,
           upd_Win, upd_bin, upd_W1, upd_b1, upd_W2, upd_b2):
    src = edge_index[0].astype(jnp.int32)
    dst = edge_index[1].astype(jnp.int32)
    wa = msg_Win[:D]
    wb = msg_Win[D:2 * D]
    wc = msg_Win[2 * D:]

    pa, pb = _tc_pre(x, wa, wb, msg_bin.reshape(1, D))
    gs, gd = _sc_gather()(pa, pb, src, dst)
    msg = _tc_msg(gs, gd, edge_attr, wc,
                  msg_W1, msg_b1.reshape(1, D), msg_W2, msg_b2.reshape(1, D))
    zeros = jnp.zeros((N_PAD, D), jnp.float32)
    part = _sc_scatter()(msg, dst, zeros)
    out = _tc_upd(part[0, :N_NODES], part[1, :N_NODES], x,
                  upd_Win, upd_bin.reshape(1, D),
                  upd_W1, upd_b1.reshape(1, D), upd_W2, upd_b2.reshape(1, D))
    return out


# speedup vs baseline: 4.1809x; 1.2198x over previous
import functools

import jax
import jax.numpy as jnp
from jax import lax
from jax.experimental import pallas as pl
from jax.experimental.pallas import tpu as pltpu
from jax.experimental.pallas import tpu_sc as plsc

N_NODES = 10000
N_EDGES = 320000
D = 128
D_EDGE = 16

NC = 2
NS = 16
NW = NC * NS

N_PAD = 10240
ROWS_PER_TILE = N_PAD // NS

E_PER_W = N_EDGES // NW
GCH = 80
G_CHUNKS = E_PER_W // GCH
VEC = 16


@functools.cache
def _sc_gather():
    mesh = plsc.VectorSubcoreMesh(core_axis_name="c", subcore_axis_name="s",
                                  num_cores=NC, num_subcores=NS)

    @functools.partial(
        pl.kernel,
        out_type=jax.ShapeDtypeStruct((N_EDGES, D), jnp.float32),
        mesh=mesh,
        scratch_types=[
            pltpu.VMEM((E_PER_W,), jnp.int32),
            pltpu.VMEM((E_PER_W,), jnp.int32),
            pltpu.VMEM((2, GCH, D), jnp.float32),
            pltpu.VMEM((2, GCH, D), jnp.float32),
            pltpu.VMEM((2, GCH, D), jnp.float32),
            pltpu.SemaphoreType.DMA,
            pltpu.SemaphoreType.DMA,
            pltpu.SemaphoreType.DMA,
            pltpu.SemaphoreType.DMA,
            pltpu.SemaphoreType.DMA,
            pltpu.SemaphoreType.DMA,
        ],
    )
    def body(pa_hbm, pb_hbm, src_hbm, dst_hbm, g_hbm,
             idxs_v, idxd_v, buf_a, buf_b, buf_o,
             sa0, sa1, sb0, sb1, sw0, sw1):
        c = lax.axis_index("c")
        s = lax.axis_index("s")
        wid = s * NC + c
        ebase = wid * E_PER_W
        sem_a = (sa0, sa1)
        sem_b = (sb0, sb1)
        sem_w = (sw0, sw1)

        pltpu.sync_copy(src_hbm.at[pl.ds(ebase, E_PER_W)], idxs_v)
        pltpu.sync_copy(dst_hbm.at[pl.ds(ebase, E_PER_W)], idxd_v)

        def fire(j, b):
            pltpu.async_copy(pa_hbm.at[idxs_v.at[pl.ds(j * GCH, GCH)]],
                             buf_a.at[b], sem_a[b])
            pltpu.async_copy(pb_hbm.at[idxd_v.at[pl.ds(j * GCH, GCH)]],
                             buf_b.at[b], sem_b[b])

        def wait_gather(j, b):
            pltpu.make_async_copy(pa_hbm.at[idxs_v.at[pl.ds(j * GCH, GCH)]],
                                  buf_a.at[b], sem_a[b]).wait()
            pltpu.make_async_copy(pb_hbm.at[idxd_v.at[pl.ds(j * GCH, GCH)]],
                                  buf_b.at[b], sem_b[b]).wait()

        def drain_write(b):
            pltpu.make_async_copy(buf_o.at[b], g_hbm.at[pl.ds(ebase, GCH)],
                                  sem_w[b]).wait()

        def add_and_write(j, b):
            @pl.loop(0, GCH, unroll=4)
            def _(r):
                for k in range(D // VEC):
                    sl = pl.ds(k * VEC, VEC)
                    buf_o[b, r, sl] = buf_a[b, r, sl] + buf_b[b, r, sl]
            pltpu.async_copy(buf_o.at[b],
                             g_hbm.at[pl.ds(ebase + j * GCH, GCH)], sem_w[b])

        fire(0, 0)
        fire(1, 1)

        @pl.loop(0, G_CHUNKS // 2)
        def _(t):
            for b in range(2):
                j = 2 * t + b
                wait_gather(j, b)

                @pl.when(t > 0)
                def _():
                    drain_write(b)

                add_and_write(j, b)

                @pl.when(j + 2 < G_CHUNKS)
                def _():
                    fire(j + 2, b)

        jl = G_CHUNKS - 1
        wait_gather(jl, 0)
        drain_write(0)
        add_and_write(jl, 0)
        drain_write(0)
        drain_write(1)

    return body


@functools.cache
def _sc_scatter():
    mesh = plsc.VectorSubcoreMesh(core_axis_name="c", subcore_axis_name="s",
                                  num_cores=NC, num_subcores=NS)

    @functools.partial(
        pl.kernel,
        out_type=jax.ShapeDtypeStruct((NC, N_PAD, D), jnp.float32),
        mesh=mesh,
        scratch_types=[
            pltpu.VMEM((G_CHUNKS, GCH), jnp.int32),
            pltpu.VMEM((3, GCH, D), jnp.float32),
            pltpu.VMEM_SHARED((N_PAD, D), jnp.float32),
            pltpu.SemaphoreType.DMA,
            pltpu.SemaphoreType.DMA,
            pltpu.SemaphoreType.DMA,
            pltpu.SemaphoreType.DMA,
            pltpu.SemaphoreType.DMA,
            pltpu.SemaphoreType.DMA,
        ],
    )
    def body(msg_hbm, dst3_hbm, zeros_hbm, part_hbm, idx_v, rows_v, acc,
             sl0, sl1, sl2, ss0, ss1, ss2):
        c = lax.axis_index("c")
        s = lax.axis_index("s")
        wid = s * NC + c
        ebase = wid * E_PER_W
        rbase = s * ROWS_PER_TILE
        sem_l = (sl0, sl1, sl2)
        sem_s = (ss0, ss1, ss2)

        pltpu.sync_copy(zeros_hbm.at[pl.ds(rbase, ROWS_PER_TILE)],
                        acc.at[pl.ds(rbase, ROWS_PER_TILE)])
        pltpu.sync_copy(dst3_hbm.at[wid], idx_v)
        plsc.subcore_barrier()

        def fire_load(j, b):
            pltpu.async_copy(msg_hbm.at[pl.ds(ebase + j * GCH, GCH)],
                             rows_v.at[b], sem_l[b])

        def wait_load(j, b):
            pltpu.make_async_copy(msg_hbm.at[pl.ds(ebase + j * GCH, GCH)],
                                  rows_v.at[b], sem_l[b]).wait()

        def drain_scatter(b):
            pltpu.make_async_copy(rows_v.at[b], acc.at[idx_v.at[0]],
                                  sem_s[b]).wait()

        def visit(j, b, bp):
            wait_load(j, b)
            pltpu.async_copy(rows_v.at[b], acc.at[idx_v.at[j]], sem_s[b],
                             add=True)

            @pl.when(j >= 1)
            def _():
                drain_scatter(bp)

            @pl.when(j + 2 < G_CHUNKS)
            def _():
                fire_load(j + 2, bp)

        fire_load(0, 0)
        fire_load(1, 1)

        @pl.loop(0, G_CHUNKS // 3)
        def _(t):
            for b in range(3):
                visit(3 * t + b, b, (b + 2) % 3)

        visit(G_CHUNKS - 2, 0, 2)
        visit(G_CHUNKS - 1, 1, 0)
        drain_scatter(1)

        plsc.subcore_barrier()
        pltpu.sync_copy(acc.at[pl.ds(rbase, ROWS_PER_TILE)],
                        part_hbm.at[c, pl.ds(rbase, ROWS_PER_TILE)])

    return body


_NB = 2000
_EB = 2560

_full = lambda shape: pl.BlockSpec(shape, lambda i: (0,) * len(shape))


def _pre_body(x_ref, wa_ref, wb_ref, bin_ref, pa_ref, pb_ref):
    xb = x_ref[...]
    pa_ref[...] = jnp.dot(xb, wa_ref[...], preferred_element_type=jnp.float32)
    pb_ref[...] = (jnp.dot(xb, wb_ref[...], preferred_element_type=jnp.float32)
                   + bin_ref[...])


def _tc_pre(x, wa, wb, b_in):
    return pl.pallas_call(
        _pre_body,
        grid=(N_NODES // _NB,),
        in_specs=[
            pl.BlockSpec((_NB, D), lambda i: (i, 0)),
            _full((D, D)), _full((D, D)), _full((1, D)),
        ],
        out_specs=[
            pl.BlockSpec((_NB, D), lambda i: (i, 0)),
            pl.BlockSpec((_NB, D), lambda i: (i, 0)),
        ],
        out_shape=[
            jax.ShapeDtypeStruct((N_NODES, D), jnp.float32),
            jax.ShapeDtypeStruct((N_NODES, D), jnp.float32),
        ],
    )(x, wa, wb, b_in)


def _msg_body(g_ref, ea_ref, wc_ref, w1_ref, b1_ref, w2_ref, b2_ref,
              out_ref):
    h = g_ref[...] + jnp.dot(
        ea_ref[...], wc_ref[...], preferred_element_type=jnp.float32)
    h = jnp.maximum(h, 0.0)
    h = h + jnp.maximum(
        jnp.dot(h, w1_ref[...], preferred_element_type=jnp.float32)
        + b1_ref[...], 0.0)
    out_ref[...] = h + jnp.maximum(
        jnp.dot(h, w2_ref[...], preferred_element_type=jnp.float32)
        + b2_ref[...], 0.0)


def _tc_msg(g, ea, wc, w1, b1, w2, b2):
    return pl.pallas_call(
        _msg_body,
        grid=(N_EDGES // _EB,),
        in_specs=[
            pl.BlockSpec((_EB, D), lambda i: (i, 0)),
            pl.BlockSpec((_EB, D_EDGE), lambda i: (i, 0)),
            _full((D_EDGE, D)), _full((D, D)), _full((1, D)),
            _full((D, D)), _full((1, D)),
        ],
        out_specs=pl.BlockSpec((_EB, D), lambda i: (i, 0)),
        out_shape=jax.ShapeDtypeStruct((N_EDGES, D), jnp.float32),
    )(g, ea, wc, w1, b1, w2, b2)


def _upd_body(p0_ref, p1_ref, x_ref, wi_ref, bi_ref, w1_ref, b1_ref,
              w2_ref, b2_ref, out_ref):
    agg = p0_ref[...] + p1_ref[...]
    h = jnp.maximum(
        jnp.dot(agg, wi_ref[...], preferred_element_type=jnp.float32)
        + bi_ref[...], 0.0)
    h = h + jnp.maximum(
        jnp.dot(h, w1_ref[...], preferred_element_type=jnp.float32)
        + b1_ref[...], 0.0)
    h = h + jnp.maximum(
        jnp.dot(h, w2_ref[...], preferred_element_type=jnp.float32)
        + b2_ref[...], 0.0)
    out_ref[...] = x_ref[...] + h


def _tc_upd(p0, p1, x, wi, bi, w1, b1, w2, b2):
    return pl.pallas_call(
        _upd_body,
        grid=(N_NODES // _NB,),
        in_specs=[
            pl.BlockSpec((_NB, D), lambda i: (i, 0)),
            pl.BlockSpec((_NB, D), lambda i: (i, 0)),
            pl.BlockSpec((_NB, D), lambda i: (i, 0)),
            _full((D, D)), _full((1, D)),
            _full((D, D)), _full((1, D)),
            _full((D, D)), _full((1, D)),
        ],
        out_specs=pl.BlockSpec((_NB, D), lambda i: (i, 0)),
        out_shape=jax.ShapeDtypeStruct((N_NODES, D), jnp.float32),
    )(p0, p1, x, wi, bi, w1, b1, w2, b2)


def kernel(x, edge_index, edge_attr,
           msg_Win, msg_bin, msg_W1, msg_b1, msg_W2, msg_b2,
           upd_Win, upd_bin, upd_W1, upd_b1, upd_W2, upd_b2):
    src = edge_index[0].astype(jnp.int32)
    dst = edge_index[1].astype(jnp.int32)
    wa = msg_Win[:D]
    wb = msg_Win[D:2 * D]
    wc = msg_Win[2 * D:]

    pa, pb = _tc_pre(x, wa, wb, msg_bin.reshape(1, D))
    g = _sc_gather()(pa, pb, src, dst)
    msg = _tc_msg(g, edge_attr, wc,
                  msg_W1, msg_b1.reshape(1, D), msg_W2, msg_b2.reshape(1, D))
    zeros = jnp.zeros((N_PAD, D), jnp.float32)
    dst3 = dst.reshape(NW, G_CHUNKS, GCH)
    part = _sc_scatter()(msg, dst3, zeros)
    out = _tc_upd(part[0, :N_NODES], part[1, :N_NODES], x,
                  upd_Win, upd_bin.reshape(1, D),
                  upd_W1, upd_b1.reshape(1, D), upd_W2, upd_b2.reshape(1, D))
    return out


# speedup vs baseline: 5.4595x; 1.3058x over previous
import functools

import jax
import jax.numpy as jnp
from jax import lax
from jax.experimental import pallas as pl
from jax.experimental.pallas import tpu as pltpu
from jax.experimental.pallas import tpu_sc as plsc

N_NODES = 10000
N_EDGES = 320000
D = 128
D_EDGE = 16

NC = 2
NS = 16
NW = NC * NS

N_PAD = 10240
ROWS_PER_TILE = N_PAD // NS

E_PER_W = N_EDGES // NW
GCH = 80
G_CHUNKS = E_PER_W // GCH
VEC = 16


@functools.cache
def _sc_gather():
    mesh = plsc.VectorSubcoreMesh(core_axis_name="c", subcore_axis_name="s",
                                  num_cores=NC, num_subcores=NS)

    @functools.partial(
        pl.kernel,
        out_type=jax.ShapeDtypeStruct((N_EDGES, D), jnp.float32),
        mesh=mesh,
        scratch_types=[
            pltpu.VMEM((E_PER_W,), jnp.int32),
            pltpu.VMEM((E_PER_W,), jnp.int32),
            pltpu.VMEM((2, GCH, D), jnp.float32),
            pltpu.VMEM((2, GCH, D), jnp.float32),
            pltpu.VMEM((2, GCH, D), jnp.float32),
            pltpu.SemaphoreType.DMA,
            pltpu.SemaphoreType.DMA,
            pltpu.SemaphoreType.DMA,
            pltpu.SemaphoreType.DMA,
            pltpu.SemaphoreType.DMA,
            pltpu.SemaphoreType.DMA,
        ],
    )
    def body(pa_hbm, pb_hbm, src_hbm, dst_hbm, g_hbm,
             idxs_v, idxd_v, buf_a, buf_b, buf_o,
             sa0, sa1, sb0, sb1, sw0, sw1):
        c = lax.axis_index("c")
        s = lax.axis_index("s")
        wid = s * NC + c
        ebase = wid * E_PER_W
        sem_a = (sa0, sa1)
        sem_b = (sb0, sb1)
        sem_w = (sw0, sw1)

        pltpu.sync_copy(src_hbm.at[pl.ds(ebase, E_PER_W)], idxs_v)
        pltpu.sync_copy(dst_hbm.at[pl.ds(ebase, E_PER_W)], idxd_v)

        def fire(j, b):
            pltpu.async_copy(pa_hbm.at[idxs_v.at[pl.ds(j * GCH, GCH)]],
                             buf_a.at[b], sem_a[b])
            pltpu.async_copy(pb_hbm.at[idxd_v.at[pl.ds(j * GCH, GCH)]],
                             buf_b.at[b], sem_b[b])

        def wait_gather(j, b):
            pltpu.make_async_copy(pa_hbm.at[idxs_v.at[pl.ds(j * GCH, GCH)]],
                                  buf_a.at[b], sem_a[b]).wait()
            pltpu.make_async_copy(pb_hbm.at[idxd_v.at[pl.ds(j * GCH, GCH)]],
                                  buf_b.at[b], sem_b[b]).wait()

        def drain_write(b):
            pltpu.make_async_copy(buf_o.at[b], g_hbm.at[pl.ds(ebase, GCH)],
                                  sem_w[b]).wait()

        def add_and_write(j, b):
            @plsc.parallel_loop(0, GCH, unroll=4)
            def _(r):
                for k in range(D // VEC):
                    sl = pl.ds(k * VEC, VEC)
                    buf_o[b, r, sl] = buf_a[b, r, sl] + buf_b[b, r, sl]
            pltpu.async_copy(buf_o.at[b],
                             g_hbm.at[pl.ds(ebase + j * GCH, GCH)], sem_w[b])

        fire(0, 0)
        fire(1, 1)

        @pl.loop(0, G_CHUNKS // 2)
        def _(t):
            for b in range(2):
                j = 2 * t + b
                wait_gather(j, b)

                @pl.when(t > 0)
                def _():
                    drain_write(b)

                add_and_write(j, b)

                @pl.when(j + 2 < G_CHUNKS)
                def _():
                    fire(j + 2, b)

        jl = G_CHUNKS - 1
        wait_gather(jl, 0)
        drain_write(0)
        add_and_write(jl, 0)
        drain_write(0)
        drain_write(1)

    return body


@functools.cache
def _sc_scatter():
    mesh = plsc.VectorSubcoreMesh(core_axis_name="c", subcore_axis_name="s",
                                  num_cores=NC, num_subcores=NS)

    @functools.partial(
        pl.kernel,
        out_type=jax.ShapeDtypeStruct((NC, N_PAD, D), jnp.float32),
        mesh=mesh,
        scratch_types=[
            pltpu.VMEM((G_CHUNKS, GCH), jnp.int32),
            pltpu.VMEM((3, GCH, D), jnp.float32),
            pltpu.VMEM_SHARED((N_PAD, D), jnp.float32),
            pltpu.SemaphoreType.DMA,
            pltpu.SemaphoreType.DMA,
            pltpu.SemaphoreType.DMA,
            pltpu.SemaphoreType.DMA,
            pltpu.SemaphoreType.DMA,
            pltpu.SemaphoreType.DMA,
        ],
    )
    def body(msg_hbm, dst3_hbm, zeros_hbm, part_hbm, idx_v, rows_v, acc,
             sl0, sl1, sl2, ss0, ss1, ss2):
        c = lax.axis_index("c")
        s = lax.axis_index("s")
        wid = s * NC + c
        ebase = wid * E_PER_W
        rbase = s * ROWS_PER_TILE
        sem_l = (sl0, sl1, sl2)
        sem_s = (ss0, ss1, ss2)

        pltpu.sync_copy(zeros_hbm.at[pl.ds(rbase, ROWS_PER_TILE)],
                        acc.at[pl.ds(rbase, ROWS_PER_TILE)])
        pltpu.sync_copy(dst3_hbm.at[wid], idx_v)
        plsc.subcore_barrier()

        def fire_load(j, b):
            pltpu.async_copy(msg_hbm.at[pl.ds(ebase + j * GCH, GCH)],
                             rows_v.at[b], sem_l[b])

        def wait_load(j, b):
            pltpu.make_async_copy(msg_hbm.at[pl.ds(ebase + j * GCH, GCH)],
                                  rows_v.at[b], sem_l[b]).wait()

        def drain_scatter(b):
            pltpu.make_async_copy(rows_v.at[b], acc.at[idx_v.at[0]],
                                  sem_s[b]).wait()

        def visit(j, b, bp):
            wait_load(j, b)
            pltpu.async_copy(rows_v.at[b], acc.at[idx_v.at[j]], sem_s[b],
                             add=True)

            @pl.when(j >= 1)
            def _():
                drain_scatter(bp)

            @pl.when(j + 2 < G_CHUNKS)
            def _():
                fire_load(j + 2, bp)

        fire_load(0, 0)
        fire_load(1, 1)

        @pl.loop(0, G_CHUNKS // 3)
        def _(t):
            for b in range(3):
                visit(3 * t + b, b, (b + 2) % 3)

        visit(G_CHUNKS - 2, 0, 2)
        visit(G_CHUNKS - 1, 1, 0)
        drain_scatter(1)

        plsc.subcore_barrier()
        pltpu.sync_copy(acc.at[pl.ds(rbase, ROWS_PER_TILE)],
                        part_hbm.at[c, pl.ds(rbase, ROWS_PER_TILE)])

    return body


_NB = 2000
_EB = 2560

_full = lambda shape: pl.BlockSpec(shape, lambda i: (0,) * len(shape))


def _pre_body(x_ref, wa_ref, wb_ref, bin_ref, pa_ref, pb_ref):
    xb = x_ref[...]
    pa_ref[...] = jnp.dot(xb, wa_ref[...], preferred_element_type=jnp.float32)
    pb_ref[...] = (jnp.dot(xb, wb_ref[...], preferred_element_type=jnp.float32)
                   + bin_ref[...])


def _tc_pre(x, wa, wb, b_in):
    return pl.pallas_call(
        _pre_body,
        grid=(N_NODES // _NB,),
        in_specs=[
            pl.BlockSpec((_NB, D), lambda i: (i, 0)),
            _full((D, D)), _full((D, D)), _full((1, D)),
        ],
        out_specs=[
            pl.BlockSpec((_NB, D), lambda i: (i, 0)),
            pl.BlockSpec((_NB, D), lambda i: (i, 0)),
        ],
        out_shape=[
            jax.ShapeDtypeStruct((N_NODES, D), jnp.float32),
            jax.ShapeDtypeStruct((N_NODES, D), jnp.float32),
        ],
    )(x, wa, wb, b_in)


def _msg_body(g_ref, ea_ref, wc_ref, w1_ref, b1_ref, w2_ref, b2_ref,
              out_ref):
    h = g_ref[...] + jnp.dot(
        ea_ref[...], wc_ref[...], preferred_element_type=jnp.float32)
    h = jnp.maximum(h, 0.0)
    h = h + jnp.maximum(
        jnp.dot(h, w1_ref[...], preferred_element_type=jnp.float32)
        + b1_ref[...], 0.0)
    out_ref[...] = h + jnp.maximum(
        jnp.dot(h, w2_ref[...], preferred_element_type=jnp.float32)
        + b2_ref[...], 0.0)


def _tc_msg(g, ea, wc, w1, b1, w2, b2):
    return pl.pallas_call(
        _msg_body,
        grid=(N_EDGES // _EB,),
        in_specs=[
            pl.BlockSpec((_EB, D), lambda i: (i, 0)),
            pl.BlockSpec((_EB, D_EDGE), lambda i: (i, 0)),
            _full((D_EDGE, D)), _full((D, D)), _full((1, D)),
            _full((D, D)), _full((1, D)),
        ],
        out_specs=pl.BlockSpec((_EB, D), lambda i: (i, 0)),
        out_shape=jax.ShapeDtypeStruct((N_EDGES, D), jnp.float32),
    )(g, ea, wc, w1, b1, w2, b2)


def _upd_body(p0_ref, p1_ref, x_ref, wi_ref, bi_ref, w1_ref, b1_ref,
              w2_ref, b2_ref, out_ref):
    agg = p0_ref[...] + p1_ref[...]
    h = jnp.maximum(
        jnp.dot(agg, wi_ref[...], preferred_element_type=jnp.float32)
        + bi_ref[...], 0.0)
    h = h + jnp.maximum(
        jnp.dot(h, w1_ref[...], preferred_element_type=jnp.float32)
        + b1_ref[...], 0.0)
    h = h + jnp.maximum(
        jnp.dot(h, w2_ref[...], preferred_element_type=jnp.float32)
        + b2_ref[...], 0.0)
    out_ref[...] = x_ref[...] + h


def _tc_upd(p0, p1, x, wi, bi, w1, b1, w2, b2):
    return pl.pallas_call(
        _upd_body,
        grid=(N_NODES // _NB,),
        in_specs=[
            pl.BlockSpec((_NB, D), lambda i: (i, 0)),
            pl.BlockSpec((_NB, D), lambda i: (i, 0)),
            pl.BlockSpec((_NB, D), lambda i: (i, 0)),
            _full((D, D)), _full((1, D)),
            _full((D, D)), _full((1, D)),
            _full((D, D)), _full((1, D)),
        ],
        out_specs=pl.BlockSpec((_NB, D), lambda i: (i, 0)),
        out_shape=jax.ShapeDtypeStruct((N_NODES, D), jnp.float32),
    )(p0, p1, x, wi, bi, w1, b1, w2, b2)


def kernel(x, edge_index, edge_attr,
           msg_Win, msg_bin, msg_W1, msg_b1, msg_W2, msg_b2,
           upd_Win, upd_bin, upd_W1, upd_b1, upd_W2, upd_b2):
    src = edge_index[0].astype(jnp.int32)
    dst = edge_index[1].astype(jnp.int32)
    wa = msg_Win[:D]
    wb = msg_Win[D:2 * D]
    wc = msg_Win[2 * D:]

    pa, pb = _tc_pre(x, wa, wb, msg_bin.reshape(1, D))
    g = _sc_gather()(pa, pb, src, dst)
    msg = _tc_msg(g, edge_attr, wc,
                  msg_W1, msg_b1.reshape(1, D), msg_W2, msg_b2.reshape(1, D))
    zeros = jnp.zeros((N_PAD, D), jnp.float32)
    dst3 = dst.reshape(NW, G_CHUNKS, GCH)
    part = _sc_scatter()(msg, dst3, zeros)
    out = _tc_upd(part[0, :N_NODES], part[1, :N_NODES], x,
                  upd_Win, upd_bin.reshape(1, D),
                  upd_W1, upd_b1.reshape(1, D), upd_W2, upd_b2.reshape(1, D))
    return out
